# Initial kernel scaffold; baseline (speedup 1.0000x reference)
import jax
import jax.numpy as jnp
from jax.experimental import pallas as pl


def kernel(x_v0, x_v1, edge_index_v0v1, edge_index_v1v0, W_rel0, b_rel0, W_root0, W_rel1, b_rel1, W_root1, W_fc, b_fc):
    raise NotImplementedError("write your pallas kernel here")


# speedup vs baseline: 5.0240x; 5.0240x over previous
import functools

import jax
import jax.numpy as jnp
from jax import lax
from jax.experimental import pallas as pl
from jax.experimental.pallas import tpu as pltpu
from jax.experimental.pallas import tpu_sc as plsc

N = 25000
D = 128
H = 64
E = 400000

NSUB = 16
CHUNK = 128
EC = 3200
EPAD = EC * CHUNK
CPT = EC // NSUB
IB = 8
NPAD = 25088
RPT = NPAD // NSUB

BN = 3136
GRID = NPAD // BN


def _tc_pre_body(x0_ref, x1_ref, wrel0_ref, wroot0_ref, b0_ref,
                 wrel1_ref, wroot1_ref, b1_ref,
                 y0_ref, y1_ref, r0_ref, r1_ref):
    dn = (((1,), (1,)), ((), ()))
    x0 = x0_ref[...]
    x1 = x1_ref[...]
    y0_ref[...] = lax.dot_general(x0, wrel0_ref[...], dn,
                                  preferred_element_type=jnp.float32)
    y1_ref[...] = lax.dot_general(x1, wrel1_ref[...], dn,
                                  preferred_element_type=jnp.float32)
    r0_ref[...] = lax.dot_general(x0, wroot0_ref[...], dn,
                                  preferred_element_type=jnp.float32) + b0_ref[...]
    r1_ref[...] = lax.dot_general(x1, wroot1_ref[...], dn,
                                  preferred_element_type=jnp.float32) + b1_ref[...]


def _tc_pre(x0, x1, wrel0, wroot0, b0, wrel1, wroot1, b1):
    row_spec = pl.BlockSpec((BN, D), lambda i: (i, 0))
    w_spec = pl.BlockSpec((H, D), lambda i: (0, 0))
    b_spec = pl.BlockSpec((1, H), lambda i: (0, 0))
    out_spec = pl.BlockSpec((BN, H), lambda i: (i, 0))
    out_shape = jax.ShapeDtypeStruct((NPAD, H), jnp.float32)
    return pl.pallas_call(
        _tc_pre_body,
        grid=(GRID,),
        in_specs=[row_spec, row_spec, w_spec, w_spec, b_spec,
                  w_spec, w_spec, b_spec],
        out_specs=[out_spec] * 4,
        out_shape=[out_shape] * 4,
    )(x0, x1, wrel0, wroot0, b0, wrel1, wroot1, b1)


def _sc_body(y0, y1, r0, r1, src0, dst0, src1, dst1,
             o0, o1, idx_s, idx_d, rows, acc, sem):
    cid = lax.axis_index("c")
    sid = lax.axis_index("s")

    def run(y, r, src, dst, out):
        base = sid * RPT
        pltpu.sync_copy(r.at[pl.ds(base, RPT)], acc.at[pl.ds(base, RPT)])
        cb = sid * CPT
        plsc.subcore_barrier()

        def outer(o, carry):
            ob = cb + o * IB
            pltpu.sync_copy(src.at[pl.ds(ob, IB)], idx_s)
            pltpu.sync_copy(dst.at[pl.ds(ob, IB)], idx_d)

            def step(j, c2):
                pltpu.async_copy(y.at[idx_s.at[j]], rows, sem).wait()
                pltpu.sync_copy(rows, acc.at[idx_d.at[j]], add=True)
                return c2

            lax.fori_loop(0, IB, step, 0)
            return carry

        lax.fori_loop(0, CPT // IB, outer, 0)
        plsc.subcore_barrier()
        pltpu.sync_copy(acc.at[pl.ds(base, RPT)], out.at[pl.ds(base, RPT)])

    @pl.when(cid == 0)
    def _():
        run(y0, r0, src0, dst0, o0)

    @pl.when(cid == 1)
    def _():
        run(y1, r1, src1, dst1, o1)


_sc_call = pl.kernel(
    _sc_body,
    out_type=[jax.ShapeDtypeStruct((NPAD, H), jnp.float32)] * 2,
    mesh=plsc.VectorSubcoreMesh(core_axis_name="c", subcore_axis_name="s"),
    scratch_types=[
        pltpu.VMEM((IB, CHUNK), jnp.int32),
        pltpu.VMEM((IB, CHUNK), jnp.int32),
        pltpu.VMEM((CHUNK, H), jnp.float32),
        pltpu.VMEM_SHARED((NPAD, H), jnp.float32),
        pltpu.SemaphoreType.DMA,
    ],
    compiler_params=pltpu.CompilerParams(use_tc_tiling_on_sc=False),
)


def _tc_post_body(s0_ref, s1_ref, wfc_ref, bfc_ref, out_ref):
    a0 = jnp.maximum(s0_ref[...], 0.0)
    a1 = jnp.maximum(s1_ref[...], 0.0)
    w = wfc_ref[...]
    out_ref[...] = (jnp.sum(a0 * w[:, :H], axis=1, keepdims=True)
                    + jnp.sum(a1 * w[:, H:], axis=1, keepdims=True)
                    + bfc_ref[...])


def _tc_post(s0, s1, wfc, bfc):
    s_spec = pl.BlockSpec((BN, H), lambda i: (i, 0))
    return pl.pallas_call(
        _tc_post_body,
        grid=(GRID,),
        in_specs=[s_spec, s_spec,
                  pl.BlockSpec((1, 2 * H), lambda i: (0, 0)),
                  pl.BlockSpec((1, 1), lambda i: (0, 0))],
        out_specs=pl.BlockSpec((BN, 1), lambda i: (i, 0)),
        out_shape=jax.ShapeDtypeStruct((NPAD, 1), jnp.float32),
    )(s0, s1, wfc, bfc)


def _prep_edges(edge_index):
    src = edge_index[0].astype(jnp.int32)
    dst = edge_index[1].astype(jnp.int32)
    pad = EPAD - E
    src = jnp.concatenate([src, jnp.zeros((pad,), jnp.int32)]).reshape(EC, CHUNK)
    dst = jnp.concatenate([dst, jnp.full((pad,), N, jnp.int32)]).reshape(EC, CHUNK)
    return src, dst


def kernel(x_v0, x_v1, edge_index_v0v1, edge_index_v1v0,
           W_rel0, b_rel0, W_root0, W_rel1, b_rel1, W_root1, W_fc, b_fc):
    src0, dst0 = _prep_edges(edge_index_v0v1)
    src1, dst1 = _prep_edges(edge_index_v1v0)

    y0, y1, r0, r1 = _tc_pre(x_v0, x_v1,
                             W_rel0, W_root0, b_rel0.reshape(1, H),
                             W_rel1, W_root1, b_rel1.reshape(1, H))

    s0, s1 = _sc_call(y0, y1, r0, r1, src0, dst0, src1, dst1)

    out = _tc_post(s0, s1, W_fc, b_fc.reshape(1, 1))
    return out[:N]


# speedup vs baseline: 5.8830x; 1.1710x over previous
import functools

import jax
import jax.numpy as jnp
from jax import lax
from jax.experimental import pallas as pl
from jax.experimental.pallas import tpu as pltpu
from jax.experimental.pallas import tpu_sc as plsc

N = 25000
D = 128
H = 64
E = 400000

NSUB = 16
CHUNK = 128
EC = 3200
EPAD = EC * CHUNK
CPT = EC // NSUB
IB = 8
NPAD = 25088
RPT = NPAD // NSUB

BN = 3136
GRID = NPAD // BN


def _tc_pre_body(x0_ref, x1_ref, wrel0_ref, wroot0_ref, b0_ref,
                 wrel1_ref, wroot1_ref, b1_ref,
                 y0_ref, y1_ref, r0_ref, r1_ref):
    dn = (((1,), (1,)), ((), ()))
    x0 = x0_ref[...]
    x1 = x1_ref[...]
    y0_ref[...] = lax.dot_general(x0, wrel0_ref[...], dn,
                                  preferred_element_type=jnp.float32)
    y1_ref[...] = lax.dot_general(x1, wrel1_ref[...], dn,
                                  preferred_element_type=jnp.float32)
    r0_ref[...] = lax.dot_general(x0, wroot0_ref[...], dn,
                                  preferred_element_type=jnp.float32) + b0_ref[...]
    r1_ref[...] = lax.dot_general(x1, wroot1_ref[...], dn,
                                  preferred_element_type=jnp.float32) + b1_ref[...]


def _tc_pre(x0, x1, wrel0, wroot0, b0, wrel1, wroot1, b1):
    row_spec = pl.BlockSpec((BN, D), lambda i: (i, 0))
    w_spec = pl.BlockSpec((H, D), lambda i: (0, 0))
    b_spec = pl.BlockSpec((1, H), lambda i: (0, 0))
    out_spec = pl.BlockSpec((BN, H), lambda i: (i, 0))
    out_shape = jax.ShapeDtypeStruct((NPAD, H), jnp.float32)
    return pl.pallas_call(
        _tc_pre_body,
        grid=(GRID,),
        in_specs=[row_spec, row_spec, w_spec, w_spec, b_spec,
                  w_spec, w_spec, b_spec],
        out_specs=[out_spec] * 4,
        out_shape=[out_shape] * 4,
    )(x0, x1, wrel0, wroot0, b0, wrel1, wroot1, b1)


def _sc_body(y0, y1, r0, r1, src0, dst0, src1, dst1,
             o0, o1, idx_s, idx_d, rows0, rows1, acc,
             gsem0, gsem1, ssem0, ssem1):
    cid = lax.axis_index("c")
    sid = lax.axis_index("s")
    rows = (rows0, rows1)
    gsem = (gsem0, gsem1)
    ssem = (ssem0, ssem1)

    def run(y, r, src, dst, out):
        base = sid * RPT
        pltpu.sync_copy(r.at[pl.ds(base, RPT)], acc.at[pl.ds(base, RPT)])
        cb = sid * CPT
        plsc.subcore_barrier()

        def outer(o, carry):
            ob = cb + o * IB
            pltpu.sync_copy(src.at[pl.ds(ob, IB)], idx_s)
            pltpu.sync_copy(dst.at[pl.ds(ob, IB)], idx_d)

            g = [None] * IB
            s = [None] * IB
            g[0] = pltpu.async_copy(y.at[idx_s.at[0]], rows[0], gsem[0])
            for j in range(IB):
                if j + 1 < IB:
                    if j >= 1:
                        s[j - 1].wait()
                    g[j + 1] = pltpu.async_copy(
                        y.at[idx_s.at[j + 1]], rows[(j + 1) % 2],
                        gsem[(j + 1) % 2])
                g[j].wait()
                s[j] = pltpu.async_copy(rows[j % 2], acc.at[idx_d.at[j]],
                                        ssem[j % 2], add=True)
            s[IB - 2].wait()
            s[IB - 1].wait()
            return carry

        lax.fori_loop(0, CPT // IB, outer, 0)
        plsc.subcore_barrier()
        pltpu.sync_copy(acc.at[pl.ds(base, RPT)], out.at[pl.ds(base, RPT)])

    @pl.when(cid == 0)
    def _():
        run(y0, r0, src0, dst0, o0)

    @pl.when(cid == 1)
    def _():
        run(y1, r1, src1, dst1, o1)


_sc_call = pl.kernel(
    _sc_body,
    out_type=[jax.ShapeDtypeStruct((NPAD, H), jnp.float32)] * 2,
    mesh=plsc.VectorSubcoreMesh(core_axis_name="c", subcore_axis_name="s"),
    scratch_types=[
        pltpu.VMEM((IB, CHUNK), jnp.int32),
        pltpu.VMEM((IB, CHUNK), jnp.int32),
        pltpu.VMEM((CHUNK, H), jnp.float32),
        pltpu.VMEM((CHUNK, H), jnp.float32),
        pltpu.VMEM_SHARED((NPAD, H), jnp.float32),
        pltpu.SemaphoreType.DMA,
        pltpu.SemaphoreType.DMA,
        pltpu.SemaphoreType.DMA,
        pltpu.SemaphoreType.DMA,
    ],
    compiler_params=pltpu.CompilerParams(use_tc_tiling_on_sc=False),
)


def _tc_post_body(s0_ref, s1_ref, wfc_ref, bfc_ref, out_ref):
    a0 = jnp.maximum(s0_ref[...], 0.0)
    a1 = jnp.maximum(s1_ref[...], 0.0)
    w = wfc_ref[...]
    out_ref[...] = (jnp.sum(a0 * w[:, :H], axis=1, keepdims=True)
                    + jnp.sum(a1 * w[:, H:], axis=1, keepdims=True)
                    + bfc_ref[...])


def _tc_post(s0, s1, wfc, bfc):
    s_spec = pl.BlockSpec((BN, H), lambda i: (i, 0))
    return pl.pallas_call(
        _tc_post_body,
        grid=(GRID,),
        in_specs=[s_spec, s_spec,
                  pl.BlockSpec((1, 2 * H), lambda i: (0, 0)),
                  pl.BlockSpec((1, 1), lambda i: (0, 0))],
        out_specs=pl.BlockSpec((BN, 1), lambda i: (i, 0)),
        out_shape=jax.ShapeDtypeStruct((NPAD, 1), jnp.float32),
    )(s0, s1, wfc, bfc)


def _prep_edges(edge_index):
    src = edge_index[0].astype(jnp.int32)
    dst = edge_index[1].astype(jnp.int32)
    pad = EPAD - E
    src = jnp.concatenate([src, jnp.zeros((pad,), jnp.int32)]).reshape(EC, CHUNK)
    dst = jnp.concatenate([dst, jnp.full((pad,), N, jnp.int32)]).reshape(EC, CHUNK)
    return src, dst


def kernel(x_v0, x_v1, edge_index_v0v1, edge_index_v1v0,
           W_rel0, b_rel0, W_root0, W_rel1, b_rel1, W_root1, W_fc, b_fc):
    src0, dst0 = _prep_edges(edge_index_v0v1)
    src1, dst1 = _prep_edges(edge_index_v1v0)

    y0, y1, r0, r1 = _tc_pre(x_v0, x_v1,
                             W_rel0, W_root0, b_rel0.reshape(1, H),
                             W_rel1, W_root1, b_rel1.reshape(1, H))

    s0, s1 = _sc_call(y0, y1, r0, r1, src0, dst0, src1, dst1)

    out = _tc_post(s0, s1, W_fc, b_fc.reshape(1, 1))
    return out[:N]
